# baseline (speedup 1.0000x reference)
import jax
import jax.numpy as jnp
from jax.experimental import pallas as pl
from jax.experimental.pallas import tpu as pltpu

_B, _N, _E = 32, 77, 1232
_C0, _H1, _H2, _H3 = 128, 64, 256, 512
_HID, _MID, _OUT = 1024, 218, 10
_CH = 7
_NCH = _N // _CH
_RB = 3


def _dot3(a, b):
    ah = a.astype(jnp.bfloat16).astype(jnp.float32)
    al = a - ah
    bh = b.astype(jnp.bfloat16).astype(jnp.float32)
    bl = b - bh
    d = lambda u, v: jnp.dot(u, v, preferred_element_type=jnp.float32)
    return d(ah, bh) + d(ah, bl) + d(al, bh)


def _fused_body(n_id_ref, ei_ref, x_hbm, L1_hbm, W1, b1, W2, b2, W3, b3,
                bl1, L2, bl2, L3, bl3, out_ref,
                xt, h3, lbuf, acc, sem_x, sem_l):

    def chunk_copies(c, slot):
        half = _CH * _H3 // 2
        return [pltpu.make_async_copy(
                    L1_hbm.at[c, pl.ds(k * half, half), :],
                    lbuf.at[slot, pl.ds(k * half, half), :],
                    sem_l.at[slot, k]) for k in range(2)]

    def start_chunk(c, slot):
        for cp in chunk_copies(c, slot):
            cp.start()

    for i in range(_N):
        pltpu.make_async_copy(
            x_hbm.at[:, pl.ds(n_id_ref[i], 1), :],
            xt.at[:, pl.ds(i, 1), :], sem_x).start()

    for c in range(_RB):
        start_chunk(c, c)

    src = ei_ref[0:1, :]
    dst = ei_ref[1:2, :]
    ion = jax.lax.broadcasted_iota(jnp.int32, (_N, _E), 0)
    S = (ion == src).astype(jnp.float32)
    D = (ion == dst).astype(jnp.float32)
    A = jax.lax.dot_general(D, S, (((1,), (1,)), ((), ())),
                            preferred_element_type=jnp.float32)
    cnt = jnp.sum(A, axis=1, keepdims=True)
    An = A / jnp.maximum(cnt, 1.0)

    def split(m):
        mh = m.astype(jnp.bfloat16)
        return mh, (m - mh.astype(jnp.float32)).astype(jnp.bfloat16)

    Anh, Anl = split(An)
    Ws = [split(W1[:_C0, :]), split(W1[_C0:, :]),
          split(W2[:_H1, :]), split(W2[_H1:, :]),
          split(W3[:_H2, :]), split(W3[_H2:, :])]

    def d(u, v):
        return jnp.dot(u, v, preferred_element_type=jnp.float32)

    def dot3(ah_al, bh_bl):
        ah, al = ah_al
        bh, bl = bh_bl
        return d(ah, bh) + d(ah, bl) + d(al, bh)

    for i in range(_N):
        pltpu.make_async_copy(
            x_hbm.at[:, pl.ds(n_id_ref[i], 1), :],
            xt.at[:, pl.ds(i, 1), :], sem_x).wait()

    def conv(h, wt, wb, br):
        hs = split(h)
        ag = dot3((Anh, Anl), hs)
        o = dot3(hs, wt) + dot3(split(ag), wb) + br[...]
        nrm = jnp.sqrt(jnp.sum(o * o, axis=-1, keepdims=True))
        o = o / jnp.maximum(nrm, 1e-12)
        return jnp.maximum(o, 0.0)

    def batch_body(b, _):
        h0 = xt[pl.ds(b, 1), :, :].reshape(_N, _C0)
        h1 = conv(h0, Ws[0], Ws[1], b1)
        h2 = conv(h1, Ws[2], Ws[3], b2)
        hb = conv(h2, Ws[4], Ws[5], b3)
        h3[pl.ds(b, 1), :, :] = hb.reshape(1, _N, _H3)
        return 0

    jax.lax.fori_loop(0, _B, batch_body, 0)

    acc[...] = jnp.zeros((_B, _HID), jnp.float32)

    def chunk_body(c, _):
        slot = jax.lax.rem(c, _RB)
        for cp in chunk_copies(c, slot):
            cp.wait()
        prod = jnp.dot(h3[:, pl.ds(c * _CH, 1), :].reshape(_B, _H3),
                       lbuf[slot, pl.ds(0, _H3), :],
                       preferred_element_type=jnp.float32)
        for i in range(1, _CH):
            prod += jnp.dot(h3[:, pl.ds(c * _CH + i, 1), :].reshape(_B, _H3),
                            lbuf[slot, pl.ds(i * _H3, _H3), :],
                            preferred_element_type=jnp.float32)
        acc[...] += prod

        @pl.when(c + _RB < _NCH)
        def _prefetch():
            start_chunk(c + _RB, slot)
        return 0

    jax.lax.fori_loop(0, _NCH, chunk_body, 0)

    hp = jax.lax.Precision.HIGHEST
    y = jnp.maximum(acc[...] + bl1[...], 0.0)
    y = jnp.maximum(
        jnp.dot(y, L2[...], preferred_element_type=jnp.float32, precision=hp)
        + bl2[...], 0.0)
    out_ref[...] = (jnp.dot(y, L3[...], preferred_element_type=jnp.float32,
                            precision=hp) + bl3[...])


def kernel(x, n_id, edge_index, W1, b1, W2, b2, W3, b3,
           L1, bl1, L2, bl2, L3, bl3):
    n_id = n_id.astype(jnp.int32)
    edge_index = edge_index.astype(jnp.int32)
    L1r = L1.reshape(_NCH, _CH * _H3, _HID)

    vmem = pl.BlockSpec(memory_space=pltpu.VMEM)
    hbm = pl.BlockSpec(memory_space=pl.ANY)
    out = pl.pallas_call(
        _fused_body,
        in_specs=[
            pl.BlockSpec(memory_space=pltpu.SMEM),
            vmem,
            hbm,
            hbm,
            vmem, vmem, vmem, vmem, vmem, vmem,
            vmem, vmem, vmem, vmem, vmem,
        ],
        out_specs=pl.BlockSpec(memory_space=pltpu.VMEM),
        out_shape=jax.ShapeDtypeStruct((_B, _OUT), jnp.float32),
        scratch_shapes=[
            pltpu.VMEM((_B, _N, _C0), jnp.float32),
            pltpu.VMEM((_B, _N, _H3), jnp.float32),
            pltpu.VMEM((_RB, _CH * _H3, _HID), jnp.float32),
            pltpu.VMEM((_B, _HID), jnp.float32),
            pltpu.SemaphoreType.DMA,
            pltpu.SemaphoreType.DMA((_RB, 2)),
        ],
    )(n_id, edge_index, x, L1r,
      W1, b1.reshape(1, -1), W2, b2.reshape(1, -1), W3, b3.reshape(1, -1),
      bl1.reshape(1, -1), L2, bl2.reshape(1, -1), L3, bl3.reshape(1, -1))
    return out


# speedup vs baseline: 1.2220x; 1.2220x over previous
import jax
import jax.numpy as jnp
from jax.experimental import pallas as pl
from jax.experimental.pallas import tpu as pltpu

_B, _N, _E = 32, 77, 1232
_C0, _H1, _H2, _H3 = 128, 64, 256, 512
_HID, _MID, _OUT = 1024, 218, 10
_CH = 7
_NCH = _N // _CH
_RB = 3
_G = 8
_NG = _B // _G
_GN = _G * _N


def _fused_body(n_id_ref, ei_ref, x_hbm, L1_hbm, W1, b1, W2, b2, W3, b3,
                bl1, L2, bl2, L3, bl3, out_ref,
                xt, h3, lbuf, acc, sem_x, sem_l):

    def start_chunk(c, slot):
        pltpu.make_async_copy(L1_hbm.at[c], lbuf.at[slot], sem_l.at[slot]).start()

    for i in range(_N):
        pltpu.make_async_copy(
            x_hbm.at[:, pl.ds(n_id_ref[i], 1), :],
            xt.at[:, pl.ds(i, 1), :], sem_x).start()

    for c in range(_RB):
        start_chunk(c, c)

    def d(u, v):
        return jnp.dot(u, v, preferred_element_type=jnp.float32)

    def split(m):
        mh = m.astype(jnp.bfloat16)
        return mh, (m - mh.astype(jnp.float32)).astype(jnp.bfloat16)

    def dot3(ah_al, bh_bl):
        ah, al = ah_al
        bh, bl = bh_bl
        return d(ah, bh) + d(ah, bl) + d(al, bh)

    src = ei_ref[0:1, :]
    dst = ei_ref[1:2, :]
    ion = jax.lax.broadcasted_iota(jnp.int32, (_N, _E), 0)
    S = (ion == src).astype(jnp.float32)
    D = (ion == dst).astype(jnp.float32)
    A = jax.lax.dot_general(D, S, (((1,), (1,)), ((), ())),
                            preferred_element_type=jnp.float32)
    cnt = jnp.sum(A, axis=1, keepdims=True)
    An = A / jnp.maximum(cnt, 1.0)
    Anh, Anl = split(An)

    rg = jax.lax.broadcasted_iota(jnp.int32, (_GN, _N), 0)
    cg = jax.lax.broadcasted_iota(jnp.int32, (_GN, _N), 1)
    Lrep = (rg % _N == cg).astype(jnp.float32)
    ri = jax.lax.broadcasted_iota(jnp.int32, (_GN, _GN), 0)
    ci = jax.lax.broadcasted_iota(jnp.int32, (_GN, _GN), 1)
    blk = ri // _N == ci // _N

    def kron(mat_bf16):
        t = d(Lrep, mat_bf16.astype(jnp.float32))
        t = jax.lax.dot_general(t, Lrep, (((1,), (1,)), ((), ())),
                                preferred_element_type=jnp.float32)
        return jnp.where(blk, t, 0.0).astype(jnp.bfloat16)

    Akh = kron(Anh)
    Akl = kron(Anl)

    Ws = [split(W1[:_C0, :]), split(W1[_C0:, :]),
          split(W2[:_H1, :]), split(W2[_H1:, :]),
          split(W3[:_H2, :]), split(W3[_H2:, :])]

    for i in range(_N):
        pltpu.make_async_copy(
            x_hbm.at[:, pl.ds(n_id_ref[i], 1), :],
            xt.at[:, pl.ds(i, 1), :], sem_x).wait()

    def conv(h, wt, wb, br):
        hs = split(h)
        pt = dot3(hs, wt)
        pb = dot3(hs, wb).astype(jnp.bfloat16)
        ag = d(Akh, pb) + d(Akl, pb)
        o = pt + ag + br[...]
        nrm = jnp.sqrt(jnp.sum(o * o, axis=-1, keepdims=True))
        o = o / jnp.maximum(nrm, 1e-12)
        return jnp.maximum(o, 0.0)

    for g in range(_NG):
        h0 = xt[pl.ds(g * _G, _G), :, :].reshape(_GN, _C0)
        h1 = conv(h0, Ws[0], Ws[1], b1)
        h2 = conv(h1, Ws[2], Ws[3], b2)
        hb = conv(h2, Ws[4], Ws[5], b3)
        h3[pl.ds(g * _G, _G), :, :] = hb.reshape(_G, _N, _H3).astype(jnp.bfloat16)

    acc[...] = jnp.zeros((_B, _HID), jnp.float32)

    for c in range(_NCH):
        slot = c % _RB
        pltpu.make_async_copy(L1_hbm.at[c], lbuf.at[slot], sem_l.at[slot]).wait()
        prod = jnp.dot(h3[:, pl.ds(c * _CH, 1), :].reshape(_B, _H3),
                       lbuf[slot, pl.ds(0, _H3), :],
                       preferred_element_type=jnp.float32)
        for i in range(1, _CH):
            prod += jnp.dot(h3[:, pl.ds(c * _CH + i, 1), :].reshape(_B, _H3),
                            lbuf[slot, pl.ds(i * _H3, _H3), :],
                            preferred_element_type=jnp.float32)
        acc[...] += prod
        if c + _RB < _NCH:
            start_chunk(c + _RB, slot)

    hp = jax.lax.Precision.HIGHEST
    y = jnp.maximum(acc[...] + bl1[...], 0.0)
    y = jnp.maximum(
        jnp.dot(y, L2[...], preferred_element_type=jnp.float32, precision=hp)
        + bl2[...], 0.0)
    out_ref[...] = (jnp.dot(y, L3[...], preferred_element_type=jnp.float32,
                            precision=hp) + bl3[...])


def kernel(x, n_id, edge_index, W1, b1, W2, b2, W3, b3,
           L1, bl1, L2, bl2, L3, bl3):
    n_id = n_id.astype(jnp.int32)
    edge_index = edge_index.astype(jnp.int32)
    L1r = L1.reshape(_NCH, _CH * _H3, _HID)

    vmem = pl.BlockSpec(memory_space=pltpu.VMEM)
    hbm = pl.BlockSpec(memory_space=pl.ANY)
    out = pl.pallas_call(
        _fused_body,
        in_specs=[
            pl.BlockSpec(memory_space=pltpu.SMEM),
            vmem,
            hbm,
            hbm,
            vmem, vmem, vmem, vmem, vmem, vmem,
            vmem, vmem, vmem, vmem, vmem,
        ],
        out_specs=pl.BlockSpec(memory_space=pltpu.VMEM),
        out_shape=jax.ShapeDtypeStruct((_B, _OUT), jnp.float32),
        scratch_shapes=[
            pltpu.VMEM((_B, _N, _C0), jnp.float32),
            pltpu.VMEM((_B, _N, _H3), jnp.bfloat16),
            pltpu.VMEM((_RB, _CH * _H3, _HID), jnp.float32),
            pltpu.VMEM((_B, _HID), jnp.float32),
            pltpu.SemaphoreType.DMA,
            pltpu.SemaphoreType.DMA((_RB,)),
        ],
    )(n_id, edge_index, x, L1r,
      W1, b1.reshape(1, -1), W2, b2.reshape(1, -1), W3, b3.reshape(1, -1),
      bl1.reshape(1, -1), L2, bl2.reshape(1, -1), L3, bl3.reshape(1, -1))
    return out
